# Initial kernel scaffold; baseline (speedup 1.0000x reference)
import jax
import jax.numpy as jnp
from jax.experimental import pallas as pl


def kernel(q_pts, s_pts, s_feats, neighb_inds, kernel_points, weights):
    raise NotImplementedError("write your pallas kernel here")


# speedup vs baseline: 1.3129x; 1.3129x over previous
import functools

import jax
import jax.numpy as jnp
from jax import lax
from jax.experimental import pallas as pl
from jax.experimental.pallas import tpu as pltpu
from jax.experimental.pallas import tpu_sc as plsc

N_PTS = 10000
H_NB = 32
K_KP = 15
K_PAD = 16
C_IN = 128
C_OUT = 128
SIGMA = 1.0

SC_CORES = 2
SC_SUBCORES = 16
SC_WORKERS = SC_CORES * SC_SUBCORES
ROWS_TOTAL = N_PTS * H_NB
ROWS_PER_W = ROWS_TOTAL // SC_WORKERS
CHUNK = 80
N_CHUNKS = ROWS_PER_W // CHUNK

MB = 200
GRID = N_PTS // MB
GROUPS = MB // 4
ROWS_PER_TILE = MB * H_NB


def _sc_gather(feats_hbm, sx_hbm, sy_hbm, sz_hbm, idx_hbm, outg_hbm, outp_hbm,
               idx_v, xt, yt, zt, fbuf, pbuf, s1):
    wid = lax.axis_index("s") * SC_CORES + lax.axis_index("c")
    base = wid * ROWS_PER_W
    pltpu.sync_copy(idx_hbm.at[pl.ds(base, ROWS_PER_W)], idx_v)
    pltpu.sync_copy(sx_hbm, xt)
    pltpu.sync_copy(sy_hbm, yt)
    pltpu.sync_copy(sz_hbm, zt)

    lane = lax.iota(jnp.int32, 16)
    row_init = jnp.where(lane == 3, 1.0, 0.0)

    def init(t, _):
        pbuf[t, :] = row_init
        return 0

    lax.fori_loop(0, CHUNK, init, 0)

    zcol = jnp.zeros((16,), jnp.int32)

    def body(j, _):
        off = j * CHUNK
        c1 = pltpu.async_copy(feats_hbm.at[idx_v.at[pl.ds(off, CHUNK)]], fbuf, s1)

        def pstep(t, _):
            iv = idx_v[pl.ds(off + t * 16, 16)]
            rows = t * 16 + lane
            plsc.store_scatter(pbuf, [rows, zcol], plsc.load_gather(xt, [iv]))
            plsc.store_scatter(pbuf, [rows, zcol + 1], plsc.load_gather(yt, [iv]))
            plsc.store_scatter(pbuf, [rows, zcol + 2], plsc.load_gather(zt, [iv]))
            return 0

        lax.fori_loop(0, CHUNK // 16, pstep, 0)
        c1.wait()
        pltpu.sync_copy(fbuf, outg_hbm.at[pl.ds(base + off, CHUNK)])
        pltpu.sync_copy(pbuf, outp_hbm.at[pl.ds(base + off, CHUNK)])
        return 0

    lax.fori_loop(0, N_CHUNKS, body, 0)


def _tc_body(g3, p2, q16, a1, a2, mask, wp, out, infl_s, wf3):
    qrep = jnp.broadcast_to(q16[:][:, None, :], (MB, H_NB, 16))
    ctr = p2[:] - qrep.reshape(ROWS_PER_TILE, 16)
    sql = ctr * ctr
    hi = lax.Precision.HIGHEST
    sq = (jnp.dot(ctr, a1[:], precision=hi, preferred_element_type=jnp.float32)
          + jnp.dot(sql, a2[:], precision=hi, preferred_element_type=jnp.float32))
    sq = jnp.maximum(sq, 0.0)
    infl = jnp.maximum(1.0 - jnp.sqrt(sq) / SIGMA, 0.0)
    infl_s[:] = infl.reshape(GROUPS, 4 * H_NB, 64) * mask[:][None]

    def grp(g, _):
        ig = infl_s[g]
        gg = g3[g]
        wf3[g] = lax.dot_general(ig, gg, (((0,), (0,)), ((), ())),
                                 preferred_element_type=jnp.float32)
        return 0

    lax.fori_loop(0, GROUPS, grp, 0)

    wf4 = wf3[:].reshape(GROUPS, K_PAD, 4, C_IN)
    acc = jnp.zeros((MB, C_OUT), jnp.float32)
    for k in range(K_PAD):
        wfk = wf4[:, k].reshape(MB, C_IN)
        acc = acc + jnp.dot(wfk, wp[k], preferred_element_type=jnp.float32)
    out[:] = acc


def kernel(q_pts, s_pts, s_feats, neighb_inds, kernel_points, weights):
    f32 = jnp.float32
    idxf = neighb_inds.reshape(-1).astype(jnp.int32)

    sx = s_pts[:, 0].astype(f32)
    sy = s_pts[:, 1].astype(f32)
    sz = s_pts[:, 2].astype(f32)
    q16 = jnp.concatenate(
        [q_pts.astype(f32), jnp.zeros((N_PTS, 13), f32)], axis=1)

    kp16 = jnp.concatenate(
        [kernel_points.astype(f32), jnp.full((1, 3), 100.0, f32)], axis=0)
    kprep = jnp.repeat(kp16, 4, axis=0)
    a1 = jnp.concatenate(
        [-2.0 * kprep.T, jnp.sum(kprep * kprep, axis=1)[None, :],
         jnp.zeros((12, 64), f32)], axis=0)
    a2 = jnp.concatenate(
        [jnp.ones((3, 64), f32), jnp.zeros((13, 64), f32)], axis=0)
    cq = lax.broadcasted_iota(jnp.int32, (4 * H_NB, 64), 0) // H_NB
    rq = lax.broadcasted_iota(jnp.int32, (4 * H_NB, 64), 1) % 4
    mask = (cq == rq).astype(f32)

    wp = jnp.concatenate(
        [weights[:, 0].astype(f32), jnp.zeros((1, C_IN, C_OUT), f32)], axis=0)

    mesh = plsc.VectorSubcoreMesh(core_axis_name="c", subcore_axis_name="s")
    sc = pl.kernel(
        _sc_gather,
        out_type=[jax.ShapeDtypeStruct((ROWS_TOTAL, C_IN), f32),
                  jax.ShapeDtypeStruct((ROWS_TOTAL, 16), f32)],
        mesh=mesh,
        scratch_types=[pltpu.VMEM((ROWS_PER_W,), jnp.int32),
                       pltpu.VMEM((N_PTS,), f32),
                       pltpu.VMEM((N_PTS,), f32),
                       pltpu.VMEM((N_PTS,), f32),
                       pltpu.VMEM((CHUNK, C_IN), f32),
                       pltpu.VMEM((CHUNK, 16), f32),
                       pltpu.SemaphoreType.DMA],
        compiler_params=pltpu.CompilerParams(needs_layout_passes=False),
    )
    gfeat, gpts = sc(s_feats.astype(f32), sx, sy, sz, idxf)

    g3 = gfeat.reshape(N_PTS // 4, 4 * H_NB, C_IN)

    out = pl.pallas_call(
        _tc_body,
        grid=(GRID,),
        in_specs=[
            pl.BlockSpec((GROUPS, 4 * H_NB, C_IN), lambda i: (i, 0, 0)),
            pl.BlockSpec((ROWS_PER_TILE, 16), lambda i: (i, 0)),
            pl.BlockSpec((MB, 16), lambda i: (i, 0)),
            pl.BlockSpec((16, 64), lambda i: (0, 0)),
            pl.BlockSpec((16, 64), lambda i: (0, 0)),
            pl.BlockSpec((4 * H_NB, 64), lambda i: (0, 0)),
            pl.BlockSpec((K_PAD, C_IN, C_OUT), lambda i: (0, 0, 0)),
        ],
        out_specs=pl.BlockSpec((MB, C_OUT), lambda i: (i, 0)),
        out_shape=jax.ShapeDtypeStruct((N_PTS, C_OUT), f32),
        scratch_shapes=[
            pltpu.VMEM((GROUPS, 4 * H_NB, 64), f32),
            pltpu.VMEM((GROUPS, 64, C_IN), f32),
        ],
    )(g3, gpts, q16, a1, a2, mask, wp)
    return out


# speedup vs baseline: 2.2094x; 1.6828x over previous
import functools

import jax
import jax.numpy as jnp
from jax import lax
from jax.experimental import pallas as pl
from jax.experimental.pallas import tpu as pltpu
from jax.experimental.pallas import tpu_sc as plsc

N_PTS = 10000
H_NB = 32
K_KP = 15
K_PAD = 16
C_IN = 128
C_OUT = 128
SIGMA = 1.0

SC_CORES = 2
SC_SUBCORES = 16
SC_WORKERS = SC_CORES * SC_SUBCORES
ROWS_TOTAL = N_PTS * H_NB
CHUNK = 128
N_CHUNKS = ROWS_TOTAL // CHUNK

MB = 200
GRID = N_PTS // MB
GROUPS = MB // 4
ROWS_PER_TILE = MB * H_NB


def _sc_gather(feats_hbm, sx_hbm, sy_hbm, sz_hbm, qx_hbm, qy_hbm, qz_hbm,
               idx_hbm, outg_hbm, outc_hbm,
               idx_v, xt, yt, zt, qxt, qyt, qzt, fbuf, cbuf, s1):
    wid = lax.axis_index("s") * SC_CORES + lax.axis_index("c")
    pltpu.sync_copy(sx_hbm, xt)
    pltpu.sync_copy(sy_hbm, yt)
    pltpu.sync_copy(sz_hbm, zt)
    pltpu.sync_copy(qx_hbm, qxt)
    pltpu.sync_copy(qy_hbm, qyt)
    pltpu.sync_copy(qz_hbm, qzt)

    lane = lax.iota(jnp.int32, 16)
    ones16 = jnp.ones((16,), jnp.float32)
    zeros16 = jnp.zeros((16,), jnp.float32)

    def init(t, _):
        cbuf[6, pl.ds(t * 16, 16)] = ones16
        cbuf[7, pl.ds(t * 16, 16)] = zeros16
        return 0

    lax.fori_loop(0, CHUNK // 16, init, 0)

    def body(j, _):
        chunk = wid + j * SC_WORKERS
        off = chunk * CHUNK
        pltpu.sync_copy(idx_hbm.at[pl.ds(off, CHUNK)], idx_v)
        c1 = pltpu.async_copy(feats_hbm.at[idx_v], fbuf, s1)

        def pstep(t, _):
            iv = idx_v[pl.ds(t * 16, 16)]
            qiv = lax.shift_right_logical(off + t * 16 + lane, 5)
            cx = plsc.load_gather(xt, [iv]) - plsc.load_gather(qxt, [qiv])
            cy = plsc.load_gather(yt, [iv]) - plsc.load_gather(qyt, [qiv])
            cz = plsc.load_gather(zt, [iv]) - plsc.load_gather(qzt, [qiv])
            sl = pl.ds(t * 16, 16)
            cbuf[0, sl] = cx
            cbuf[1, sl] = cy
            cbuf[2, sl] = cz
            cbuf[3, sl] = cx * cx
            cbuf[4, sl] = cy * cy
            cbuf[5, sl] = cz * cz
            return 0

        lax.fori_loop(0, CHUNK // 16, pstep, 0)
        c1.wait()
        pltpu.sync_copy(fbuf, outg_hbm.at[pl.ds(off, CHUNK)])
        pltpu.sync_copy(cbuf, outc_hbm.at[:, pl.ds(off, CHUNK)])
        return 0

    nj = jnp.where(wid < N_CHUNKS % SC_WORKERS,
                   N_CHUNKS // SC_WORKERS + 1, N_CHUNKS // SC_WORKERS)
    lax.fori_loop(0, nj, body, 0)


def _tc_body(g3, ct, bm, mask, wp, out, infl_s, wf3):
    sq = lax.dot_general(bm[:], ct[:], (((1,), (0,)), ((), ())),
                         precision=lax.Precision.HIGHEST,
                         preferred_element_type=jnp.float32)
    sq = jnp.maximum(sq, 0.0)
    infl_s[:] = jnp.maximum(1.0 - jnp.sqrt(sq) / SIGMA, 0.0) * mask[:]

    def grp(g, _):
        ig = infl_s[:, pl.ds(g * 128, 128)]
        gg = g3[g]
        wf3[g] = jnp.dot(ig, gg, preferred_element_type=jnp.float32)
        return 0

    lax.fori_loop(0, GROUPS, grp, 0)

    wf4 = wf3[:].reshape(GROUPS, K_PAD, 4, C_IN)
    acc = jnp.zeros((MB, C_OUT), jnp.float32)
    for k in range(K_PAD):
        wfk = wf4[:, k].reshape(MB, C_IN)
        acc = acc + jnp.dot(wfk, wp[k], preferred_element_type=jnp.float32)
    out[:] = acc


def kernel(q_pts, s_pts, s_feats, neighb_inds, kernel_points, weights):
    f32 = jnp.float32
    idxf = neighb_inds.reshape(-1).astype(jnp.int32)

    sx = s_pts[:, 0].astype(f32)
    sy = s_pts[:, 1].astype(f32)
    sz = s_pts[:, 2].astype(f32)
    qx = q_pts[:, 0].astype(f32)
    qy = q_pts[:, 1].astype(f32)
    qz = q_pts[:, 2].astype(f32)

    kp16 = jnp.concatenate(
        [kernel_points.astype(f32), jnp.full((1, 3), 100.0, f32)], axis=0)
    kprep = jnp.repeat(kp16, 4, axis=0)
    bm = jnp.concatenate(
        [-2.0 * kprep, jnp.ones((64, 3), f32),
         jnp.sum(kprep * kprep, axis=1)[:, None], jnp.zeros((64, 1), f32)],
        axis=1)
    cq = (lax.broadcasted_iota(jnp.int32, (64, ROWS_PER_TILE), 1) % 128) // H_NB
    rq = lax.broadcasted_iota(jnp.int32, (64, ROWS_PER_TILE), 0) % 4
    mask = (cq == rq).astype(f32)

    wp = jnp.concatenate(
        [weights[:, 0].astype(f32), jnp.zeros((1, C_IN, C_OUT), f32)], axis=0)

    mesh = plsc.VectorSubcoreMesh(core_axis_name="c", subcore_axis_name="s")
    sc = pl.kernel(
        _sc_gather,
        out_type=[jax.ShapeDtypeStruct((ROWS_TOTAL, C_IN), f32),
                  jax.ShapeDtypeStruct((8, ROWS_TOTAL), f32)],
        mesh=mesh,
        scratch_types=[pltpu.VMEM((CHUNK,), jnp.int32),
                       pltpu.VMEM((N_PTS,), f32),
                       pltpu.VMEM((N_PTS,), f32),
                       pltpu.VMEM((N_PTS,), f32),
                       pltpu.VMEM((N_PTS,), f32),
                       pltpu.VMEM((N_PTS,), f32),
                       pltpu.VMEM((N_PTS,), f32),
                       pltpu.VMEM((CHUNK, C_IN), f32),
                       pltpu.VMEM((8, CHUNK), f32),
                       pltpu.SemaphoreType.DMA],
        compiler_params=pltpu.CompilerParams(needs_layout_passes=False),
    )
    gfeat, ctall = sc(s_feats.astype(f32), sx, sy, sz, qx, qy, qz, idxf)

    g3 = gfeat.reshape(N_PTS // 4, 4 * H_NB, C_IN)

    out = pl.pallas_call(
        _tc_body,
        grid=(GRID,),
        in_specs=[
            pl.BlockSpec((GROUPS, 4 * H_NB, C_IN), lambda i: (i, 0, 0)),
            pl.BlockSpec((8, ROWS_PER_TILE), lambda i: (0, i)),
            pl.BlockSpec((64, 8), lambda i: (0, 0)),
            pl.BlockSpec((64, ROWS_PER_TILE), lambda i: (0, 0)),
            pl.BlockSpec((K_PAD, C_IN, C_OUT), lambda i: (0, 0, 0)),
        ],
        out_specs=pl.BlockSpec((MB, C_OUT), lambda i: (i, 0)),
        out_shape=jax.ShapeDtypeStruct((N_PTS, C_OUT), f32),
        scratch_shapes=[
            pltpu.VMEM((64, ROWS_PER_TILE), f32),
            pltpu.VMEM((GROUPS, 64, C_IN), f32),
        ],
    )(g3, ctall, bm, mask, wp)
    return out


# speedup vs baseline: 2.2599x; 1.0229x over previous
import functools

import jax
import jax.numpy as jnp
from jax import lax
from jax.experimental import pallas as pl
from jax.experimental.pallas import tpu as pltpu
from jax.experimental.pallas import tpu_sc as plsc

N_PTS = 10000
H_NB = 32
K_KP = 15
K_PAD = 16
C_IN = 128
C_OUT = 128
SIGMA = 1.0

SC_CORES = 2
SC_SUBCORES = 16
SC_WORKERS = SC_CORES * SC_SUBCORES
ROWS_TOTAL = N_PTS * H_NB
CHUNK = 128
N_CHUNKS = ROWS_TOTAL // CHUNK

MB = 400
GRID = N_PTS // MB
GROUPS = MB // 4
ROWS_PER_TILE = MB * H_NB


def _sc_gather(feats_hbm, sx_hbm, sy_hbm, sz_hbm, qx_hbm, qy_hbm, qz_hbm,
               idx_hbm, outg_hbm, outc_hbm,
               idx_v, xt, yt, zt, qxt, qyt, qzt, fbuf, cbuf, s1):
    wid = lax.axis_index("s") * SC_CORES + lax.axis_index("c")
    pltpu.sync_copy(sx_hbm, xt)
    pltpu.sync_copy(sy_hbm, yt)
    pltpu.sync_copy(sz_hbm, zt)
    pltpu.sync_copy(qx_hbm, qxt)
    pltpu.sync_copy(qy_hbm, qyt)
    pltpu.sync_copy(qz_hbm, qzt)

    lane = lax.iota(jnp.int32, 16)
    ones16 = jnp.ones((16,), jnp.float32)
    zeros16 = jnp.zeros((16,), jnp.float32)

    def init(t, _):
        cbuf[6, pl.ds(t * 16, 16)] = ones16
        cbuf[7, pl.ds(t * 16, 16)] = zeros16
        return 0

    lax.fori_loop(0, CHUNK // 16, init, 0)

    def body(j, _):
        chunk = wid + j * SC_WORKERS
        off = chunk * CHUNK
        pltpu.sync_copy(idx_hbm.at[pl.ds(off, CHUNK)], idx_v)
        c1 = pltpu.async_copy(feats_hbm.at[idx_v], fbuf, s1)

        def pstep(t, _):
            iv = idx_v[pl.ds(t * 16, 16)]
            qiv = lax.shift_right_logical(off + t * 16 + lane, 5)
            cx = plsc.load_gather(xt, [iv]) - plsc.load_gather(qxt, [qiv])
            cy = plsc.load_gather(yt, [iv]) - plsc.load_gather(qyt, [qiv])
            cz = plsc.load_gather(zt, [iv]) - plsc.load_gather(qzt, [qiv])
            sl = pl.ds(t * 16, 16)
            cbuf[0, sl] = cx
            cbuf[1, sl] = cy
            cbuf[2, sl] = cz
            cbuf[3, sl] = cx * cx
            cbuf[4, sl] = cy * cy
            cbuf[5, sl] = cz * cz
            return 0

        lax.fori_loop(0, CHUNK // 16, pstep, 0)
        c1.wait()
        pltpu.sync_copy(fbuf, outg_hbm.at[pl.ds(off, CHUNK)])
        pltpu.sync_copy(cbuf, outc_hbm.at[:, pl.ds(off, CHUNK)])
        return 0

    nj = jnp.where(wid < N_CHUNKS % SC_WORKERS,
                   N_CHUNKS // SC_WORKERS + 1, N_CHUNKS // SC_WORKERS)
    lax.fori_loop(0, nj, body, 0)


def _tc_body(g3, ct, bm, mask, wp, out, infl_s, wf3):
    sq = lax.dot_general(bm[:], ct[:], (((1,), (0,)), ((), ())),
                         precision=lax.Precision.HIGHEST,
                         preferred_element_type=jnp.float32)
    sq = jnp.maximum(sq, 0.0)
    infl_s[:] = jnp.maximum(1.0 - jnp.sqrt(sq) / SIGMA, 0.0)
    maskb = mask[:]

    def grp(g, _):
        ig = infl_s[:, pl.ds(g * 128, 128)] * maskb
        gg = g3[g]
        wf3[g] = jnp.dot(ig, gg, preferred_element_type=jnp.float32)
        return 0

    lax.fori_loop(0, GROUPS, grp, 0)

    wf4 = wf3[:].reshape(GROUPS, K_PAD, 4, C_IN)
    acc = jnp.zeros((MB, C_OUT), jnp.float32)
    for k in range(K_PAD):
        wfk = wf4[:, k].reshape(MB, C_IN)
        acc = acc + jnp.dot(wfk, wp[k], preferred_element_type=jnp.float32)
    out[:] = acc


def kernel(q_pts, s_pts, s_feats, neighb_inds, kernel_points, weights):
    f32 = jnp.float32
    idxf = neighb_inds.reshape(-1).astype(jnp.int32)

    sx = s_pts[:, 0].astype(f32)
    sy = s_pts[:, 1].astype(f32)
    sz = s_pts[:, 2].astype(f32)
    qx = q_pts[:, 0].astype(f32)
    qy = q_pts[:, 1].astype(f32)
    qz = q_pts[:, 2].astype(f32)

    kp16 = jnp.concatenate(
        [kernel_points.astype(f32), jnp.full((1, 3), 100.0, f32)], axis=0)
    kprep = jnp.repeat(kp16, 4, axis=0)
    bm = jnp.concatenate(
        [-2.0 * kprep, jnp.ones((64, 3), f32),
         jnp.sum(kprep * kprep, axis=1)[:, None], jnp.zeros((64, 1), f32)],
        axis=1)
    cq = lax.broadcasted_iota(jnp.int32, (64, 128), 1) // H_NB
    rq = lax.broadcasted_iota(jnp.int32, (64, 128), 0) % 4
    mask = (cq == rq).astype(f32)

    wp = jnp.concatenate(
        [weights[:, 0].astype(f32), jnp.zeros((1, C_IN, C_OUT), f32)], axis=0)

    mesh = plsc.VectorSubcoreMesh(core_axis_name="c", subcore_axis_name="s")
    sc = pl.kernel(
        _sc_gather,
        out_type=[jax.ShapeDtypeStruct((ROWS_TOTAL, C_IN), f32),
                  jax.ShapeDtypeStruct((8, ROWS_TOTAL), f32)],
        mesh=mesh,
        scratch_types=[pltpu.VMEM((CHUNK,), jnp.int32),
                       pltpu.VMEM((N_PTS,), f32),
                       pltpu.VMEM((N_PTS,), f32),
                       pltpu.VMEM((N_PTS,), f32),
                       pltpu.VMEM((N_PTS,), f32),
                       pltpu.VMEM((N_PTS,), f32),
                       pltpu.VMEM((N_PTS,), f32),
                       pltpu.VMEM((CHUNK, C_IN), f32),
                       pltpu.VMEM((8, CHUNK), f32),
                       pltpu.SemaphoreType.DMA],
        compiler_params=pltpu.CompilerParams(needs_layout_passes=False),
    )
    gfeat, ctall = sc(s_feats, sx, sy, sz, qx, qy, qz, idxf)

    g3 = gfeat.reshape(N_PTS // 4, 4 * H_NB, C_IN)

    out = pl.pallas_call(
        _tc_body,
        grid=(GRID,),
        in_specs=[
            pl.BlockSpec((GROUPS, 4 * H_NB, C_IN), lambda i: (i, 0, 0)),
            pl.BlockSpec((8, ROWS_PER_TILE), lambda i: (0, i)),
            pl.BlockSpec((64, 8), lambda i: (0, 0)),
            pl.BlockSpec((64, 128), lambda i: (0, 0)),
            pl.BlockSpec((K_PAD, C_IN, C_OUT), lambda i: (0, 0, 0)),
        ],
        out_specs=pl.BlockSpec((MB, C_OUT), lambda i: (i, 0)),
        out_shape=jax.ShapeDtypeStruct((N_PTS, C_OUT), f32),
        scratch_shapes=[
            pltpu.VMEM((64, ROWS_PER_TILE), f32),
            pltpu.VMEM((GROUPS, 64, C_IN), f32),
        ],
    )(g3, ctall, bm, mask, wp)
    return out
